# baseline (speedup 1.0000x reference)
import functools

import jax
import jax.numpy as jnp
from jax import lax
from jax.experimental import pallas as pl
from jax.experimental.pallas import tpu as pltpu
from jax.experimental.pallas import tpu_sc as plsc


def _positional_table(seq_len, dim):
    even_i = jnp.arange(0, dim, 2).astype(jnp.float32)
    denominator = jnp.power(10000.0, even_i / dim)
    position = jnp.arange(seq_len, dtype=jnp.float32).reshape(seq_len, 1)
    even_pe = jnp.sin(position / denominator)
    odd_pe = jnp.cos(position / denominator)
    return jnp.stack([even_pe, odd_pe], axis=2).reshape(seq_len, dim)


def kernel(x, embedding):
    batch, seq = x.shape
    vocab, dim = embedding.shape

    info = plsc.get_sparse_core_info()
    nw = info.num_cores * info.num_subcores
    bpw = batch // nw
    n_pairs = seq // 2

    xt = x.T.astype(jnp.int32)
    peb = jnp.broadcast_to(
        _positional_table(seq, dim)[:, :, None], (seq, dim, 16)
    ).reshape(seq * dim * 16)

    dh_n = dim // 8
    blk = batch // 128

    mesh = plsc.VectorSubcoreMesh(core_axis_name="c", subcore_axis_name="s")

    @functools.partial(
        pl.kernel,
        mesh=mesh,
        out_type=jax.ShapeDtypeStruct((seq, dh_n, blk, 8, 128), jnp.float32),
        compiler_params=pltpu.CompilerParams(
            use_tc_tiling_on_sc=False, needs_layout_passes=False
        ),
        scratch_types=[
            pltpu.VMEM((bpw,), jnp.int32),
            pltpu.VMEM((bpw,), jnp.int32),
            pltpu.VMEM((dim * 16,), jnp.float32),
            pltpu.VMEM((dim * 16,), jnp.float32),
            pltpu.VMEM((bpw, dim), jnp.float32),
            pltpu.VMEM((bpw, dim), jnp.float32),
            pltpu.VMEM((dh_n, 8, 128), jnp.float32),
            pltpu.VMEM((dh_n, 8, 128), jnp.float32),
            pltpu.SemaphoreType.DMA,
            pltpu.SemaphoreType.DMA,
            pltpu.SemaphoreType.DMA,
            pltpu.SemaphoreType.DMA,
            pltpu.SemaphoreType.DMA,
            pltpu.SemaphoreType.DMA,
        ],
    )
    def sc_kernel(xt_hbm, emb_hbm, peb_hbm, out_hbm,
                  idx0, idx1, pe0, pe1, rows0, rows1, ot0, ot1,
                  semi0, semi1, semg0, semg1, sems0, sems1):
        cid = lax.axis_index("c")
        sid = lax.axis_index("s")
        wid = sid * info.num_cores + cid
        b0 = wid * bpw

        def in_copies(s, idx_v, pe_v, semi):
            return [
                pltpu.make_async_copy(
                    xt_hbm.at[s, pl.ds(b0, bpw)], idx_v, semi
                ),
                pltpu.make_async_copy(
                    peb_hbm.at[pl.ds(s * dim * 16, dim * 16)], pe_v, semi
                ),
            ]

        def gather_copy(idx_v, rows_v, semg):
            return pltpu.make_async_copy(emb_hbm.at[idx_v], rows_v, semg)

        def out_copies(s, ot_v, sems):
            return [
                pltpu.make_async_copy(
                    ot_v.at[dh],
                    out_hbm.at[s, dh, wid],
                    sems,
                )
                for dh in range(dh_n)
            ]

        iota16 = lax.iota(jnp.int32, 16)

        def compute(rows_v, pe_v, ot_v):
            @plsc.parallel_loop(0, dim * 8, unroll=8)
            def _(t):
                d = t >> 3
                bg = t & 7
                pv = pe_v[pl.ds(d * 16, 16)]
                vals = plsc.load_gather(
                    rows_v, [iota16 + bg * 16, jnp.broadcast_to(d, (16,))]
                )
                ot_v[d >> 3, d & 7, pl.ds(bg * 16, 16)] = vals + pv

        for cp in in_copies(0, idx0, pe0, semi0):
            cp.start()
        for cp in in_copies(0, idx0, pe0, semi0):
            cp.wait()
        gather_copy(idx0, rows0, semg0).start()
        for cp in in_copies(1, idx1, pe1, semi1):
            cp.start()
        for cp in in_copies(1, idx1, pe1, semi1):
            cp.wait()
        gather_copy(idx1, rows1, semg1).start()

        def pair_body(p, carry):
            s_a = 2 * p
            s_b = s_a + 1

            gather_copy(idx0, rows0, semg0).wait()
            compute(rows0, pe0, ot0)
            for cp in out_copies(s_a, ot0, sems0):
                cp.start()

            @pl.when(p < n_pairs - 1)
            def _():
                for cp in in_copies(s_a + 2, idx0, pe0, semi0):
                    cp.start()

            gather_copy(idx1, rows1, semg1).wait()
            compute(rows1, pe1, ot1)
            for cp in out_copies(s_b, ot1, sems1):
                cp.start()

            @pl.when(p < n_pairs - 1)
            def _():
                for cp in in_copies(s_b + 2, idx1, pe1, semi1):
                    cp.start()

            for cp in out_copies(s_a, ot0, sems0):
                cp.wait()

            @pl.when(p < n_pairs - 1)
            def _():
                for cp in in_copies(s_a + 2, idx0, pe0, semi0):
                    cp.wait()
                gather_copy(idx0, rows0, semg0).start()

            for cp in out_copies(s_b, ot1, sems1):
                cp.wait()

            @pl.when(p < n_pairs - 1)
            def _():
                for cp in in_copies(s_b + 2, idx1, pe1, semi1):
                    cp.wait()
                gather_copy(idx1, rows1, semg1).start()

            return carry

        lax.fori_loop(0, n_pairs, pair_body, 0)

    raw = sc_kernel(xt, embedding, peb)
    return raw.transpose(2, 4, 0, 1, 3).reshape(batch, seq, dim)


# speedup vs baseline: 1.2510x; 1.2510x over previous
import functools

import jax
import jax.numpy as jnp
from jax import lax
from jax.experimental import pallas as pl
from jax.experimental.pallas import tpu as pltpu
from jax.experimental.pallas import tpu_sc as plsc


def _positional_table(seq_len, dim):
    even_i = jnp.arange(0, dim, 2).astype(jnp.float32)
    denominator = jnp.power(10000.0, even_i / dim)
    position = jnp.arange(seq_len, dtype=jnp.float32).reshape(seq_len, 1)
    even_pe = jnp.sin(position / denominator)
    odd_pe = jnp.cos(position / denominator)
    return jnp.stack([even_pe, odd_pe], axis=2).reshape(seq_len, dim)


def kernel(x, embedding):
    batch, seq = x.shape
    vocab, dim = embedding.shape

    info = plsc.get_sparse_core_info()
    num_workers = info.num_cores * info.num_subcores

    rows_per_chunk = 2
    chunk = rows_per_chunk * seq
    per_worker = batch // num_workers
    n_chunks = per_worker // rows_per_chunk
    n_pairs = n_chunks // 2

    splits = []
    off = 0
    while off < seq:
        size = min(128, seq - off)
        splits.append((off, size))
        off += size

    xi = x.astype(jnp.int32)
    pe1 = _positional_table(seq, dim).reshape(seq * dim)

    mesh = plsc.VectorSubcoreMesh(core_axis_name="c", subcore_axis_name="s")

    @functools.partial(
        pl.kernel,
        mesh=mesh,
        out_type=jax.ShapeDtypeStruct((batch, seq, dim), jnp.float32),
        compiler_params=pltpu.CompilerParams(use_tc_tiling_on_sc=False),
        scratch_types=[
            pltpu.VMEM((rows_per_chunk, seq), jnp.int32),
            pltpu.VMEM((rows_per_chunk, seq), jnp.int32),
            pltpu.VMEM((rows_per_chunk, seq, dim), jnp.float32),
            pltpu.VMEM((rows_per_chunk, seq, dim), jnp.float32),
            pltpu.VMEM((seq * dim,), jnp.float32),
            pltpu.SemaphoreType.DMA,
            pltpu.SemaphoreType.DMA,
            pltpu.SemaphoreType.DMA,
            pltpu.SemaphoreType.DMA,
        ],
    )
    def sc_kernel(xi_hbm, emb_hbm, pe_hbm, out_hbm,
                  idx0, idx1, rows0, rows1, pe_v,
                  semg0, semg1, sems0, sems1):
        cid = lax.axis_index("c")
        sid = lax.axis_index("s")
        wid = sid * info.num_cores + cid
        pltpu.sync_copy(pe_hbm, pe_v)
        row_base = wid * per_worker

        def load_idx(ch, idx_v):
            pltpu.sync_copy(
                xi_hbm.at[pl.ds(row_base + ch * rows_per_chunk, rows_per_chunk)],
                idx_v,
            )

        def gather_copies(idx_v, rows_v, semg):
            return [
                pltpu.make_async_copy(
                    emb_hbm.at[idx_v.at[i, pl.ds(off, size)]],
                    rows_v.at[i, pl.ds(off, size)],
                    semg,
                )
                for i in range(rows_per_chunk)
                for off, size in splits
            ]

        def add_pe(rows_v):
            for i in range(rows_per_chunk):
                @plsc.parallel_loop(0, seq, unroll=4)
                def _(r):
                    for dg in range(dim // 16):
                        sl = pl.ds(dg * 16, 16)
                        rows_v[i, r, sl] = rows_v[i, r, sl] + pe_v[
                            pl.ds(r * dim + dg * 16, 16)
                        ]

        def store_copy(ch, rows_v, sems):
            return pltpu.make_async_copy(
                rows_v,
                out_hbm.at[pl.ds(row_base + ch * rows_per_chunk, rows_per_chunk)],
                sems,
            )

        load_idx(0, idx0)
        for cp in gather_copies(idx0, rows0, semg0):
            cp.start()
        load_idx(1, idx1)
        for cp in gather_copies(idx1, rows1, semg1):
            cp.start()

        def pair_body(p, carry):
            ch_a = 2 * p
            ch_b = ch_a + 1
            for cp in gather_copies(idx0, rows0, semg0):
                cp.wait()
            add_pe(rows0)
            store_copy(ch_a, rows0, sems0).start()
            for cp in gather_copies(idx1, rows1, semg1):
                cp.wait()
            add_pe(rows1)
            store_copy(ch_b, rows1, sems1).start()
            store_copy(ch_a, rows0, sems0).wait()

            @pl.when(p < n_pairs - 1)
            def _():
                load_idx(ch_a + 2, idx0)
                for cp in gather_copies(idx0, rows0, semg0):
                    cp.start()

            store_copy(ch_b, rows1, sems1).wait()

            @pl.when(p < n_pairs - 1)
            def _():
                load_idx(ch_b + 2, idx1)
                for cp in gather_copies(idx1, rows1, semg1):
                    cp.start()

            return carry

        lax.fori_loop(0, n_pairs, pair_body, 0)

    return sc_kernel(xi, embedding, pe1)
